# baseline (speedup 1.0000x reference)
import functools

import jax
import jax.numpy as jnp
from jax import lax
from jax.experimental import pallas as pl
from jax.experimental.pallas import tpu as pltpu
from jax.experimental.pallas import tpu_sc as plsc

_NC = 2
_NS = 16
_NW = _NC * _NS
_BC = 256


@jax.jit
def _embed_native(idx_t, table):
    L, B = idx_t.shape
    V, D = table.shape
    n_bch = B // _BC
    n_units = L * n_bch
    per_w = n_units // _NW
    mesh = plsc.VectorSubcoreMesh(
        core_axis_name="c", subcore_axis_name="s",
        num_cores=_NC, num_subcores=_NS)

    @functools.partial(
        pl.kernel,
        out_type=jax.ShapeDtypeStruct((L, D, B), jnp.float32),
        mesh=mesh,
        scratch_types=[
            pltpu.VMEM((_BC,), jnp.int32),
            pltpu.VMEM((_BC,), jnp.int32),
            pltpu.VMEM((_BC, D), jnp.float32),
            pltpu.VMEM((_BC, D), jnp.float32),
            pltpu.VMEM((D, _BC), jnp.float32),
            pltpu.VMEM((D, _BC), jnp.float32),
            pltpu.SemaphoreType.DMA,
            pltpu.SemaphoreType.DMA,
            pltpu.SemaphoreType.DMA,
            pltpu.SemaphoreType.DMA,
            pltpu.SemaphoreType.DMA,
            pltpu.SemaphoreType.DMA,
        ],
        compiler_params=pltpu.CompilerParams(
            use_tc_tiling_on_sc=False, needs_layout_passes=False),
    )
    def k(idx_hbm, tab_hbm, out_hbm,
          idx0, idx1, rows0, rows1, tout0, tout1,
          is0, is1, gs0, gs1, os0, os1):
        wid = lax.axis_index("s") * _NC + lax.axis_index("c")
        u0 = wid * per_w
        idxv = (idx0, idx1)
        rowsv = (rows0, rows1)
        toutv = (tout0, tout1)
        isem = (is0, is1)
        gsem = (gs0, gs1)
        osem = (os0, os1)

        def unit_lb(j):
            u = u0 + j
            return u >> 4, u & (n_bch - 1)

        def idx_src(j):
            l, bch = unit_lb(j)
            return idx_hbm.at[l, pl.ds(bch * _BC, _BC)]

        def out_dst(j):
            l, bch = unit_lb(j)
            return out_hbm.at[l, :, pl.ds(bch * _BC, _BC)]

        def start_idx(j, b):
            pltpu.async_copy(idx_src(j), idxv[b], isem[b])

        def wait_idx(j, b):
            pltpu.make_async_copy(idx_src(j), idxv[b], isem[b]).wait()

        def start_gather(b):
            pltpu.async_copy(tab_hbm.at[idxv[b]], rowsv[b], gsem[b])

        def wait_gather(b):
            pltpu.make_async_copy(tab_hbm.at[idxv[b]], rowsv[b],
                                  gsem[b]).wait()

        def start_out(j, b):
            pltpu.async_copy(toutv[b], out_dst(j), osem[b])

        def wait_out(j, b):
            pltpu.make_async_copy(toutv[b], out_dst(j), osem[b]).wait()

        d_base = [lax.iota(jnp.int32, 16) + (16 * g) for g in range(D // 16)]

        def transpose(b):
            rows = rowsv[b]
            tout = toutv[b]

            @pl.loop(0, _BC, unroll=8)
            def _(bi):
                col = jnp.full((16,), bi, jnp.int32)
                for g in range(D // 16):
                    vec = rows[bi, pl.ds(16 * g, 16)]
                    plsc.store_scatter(tout, [d_base[g], col], vec)

        start_idx(0, 0)
        start_idx(1, 1)
        wait_idx(0, 0)
        start_gather(0)

        @pl.loop(0, per_w, step=2)
        def _(j0):
            for t in range(2):
                j = j0 + t
                b = t
                ob = 1 - t
                wait_gather(b)

                @pl.when(j + 2 < per_w)
                def _():
                    start_idx(j + 2, b)

                @pl.when(j + 1 < per_w)
                def _():
                    wait_idx(j + 1, ob)
                    start_gather(ob)

                @pl.when(j >= 2)
                def _():
                    wait_out(j - 2, b)

                transpose(b)
                start_out(j, b)

        wait_out(per_w - 2, 0)
        wait_out(per_w - 1, 1)

    return k(idx_t, table)


def kernel(indices, table):
    out_t = _embed_native(indices.T, table)
    return jnp.transpose(out_t, (2, 0, 1))
